# baseline (speedup 1.0000x reference)
import functools

import jax
import jax.numpy as jnp
from jax import lax
from jax.experimental import pallas as pl
from jax.experimental.pallas import tpu as pltpu
from jax.experimental.pallas import tpu_sc as plsc

_NUM_HARD = 2
_SIGN = -2147483648
_MANT = 2147483647

_N_TILES = 16
_PER_TILE = 62528
_N_PAD = _N_TILES * _PER_TILE
_CHUNKS = _PER_TILE // 16


def _sc_select_body(k0, neg_hbm, t_out, data_v, hist_v, loc_v, shared_v,
                    merge_v, tvec_v):
    cid = lax.axis_index("c")
    sid = lax.axis_index("s")
    pltpu.sync_copy(neg_hbm.at[pl.ds(sid * _PER_TILE, _PER_TILE)], data_v)
    lane = lax.iota(jnp.int32, 16)
    ones = jnp.ones((16,), jnp.int32)

    def zero_hist():
        def zb(i, _):
            hist_v[pl.ds(i * 16, 16)] = jnp.zeros((16,), jnp.int32)
            return 0
        lax.fori_loop(0, 256, zb, 0)

    def reduce16(src_v):
        def lr(cb, _):
            def rr(r, acc):
                return acc + src_v[pl.ds(r * 256 + cb * 16, 16)]
            acc = lax.fori_loop(0, 16, rr, jnp.zeros((16,), jnp.int32))
            loc_v[pl.ds(cb * 16, 16)] = acc
            return 0
        lax.fori_loop(0, 16, lr, 0)

    def merge_and_scan(kcur):
        reduce16(hist_v)
        pltpu.sync_copy(loc_v, shared_v.at[pl.ds(sid * 256, 256)])
        plsc.subcore_barrier()
        pltpu.sync_copy(shared_v, merge_v)
        plsc.subcore_barrier()
        reduce16(merge_v)

        def sc(i, carry):
            run, bstar, astar = carry
            cc = 15 - i
            h = loc_v[pl.ds(cc * 16, 16)]
            tot = jnp.sum(h)
            above = (run + tot) - jnp.cumsum(h)
            maska = above < kcur
            cntm = jnp.sum(maska.astype(jnp.int32))
            found = cntm > 0
            j0 = 16 - cntm
            aat = jnp.sum(jnp.where(lane == j0, above, 0))
            bstar = jnp.where(found, cc * 16 + j0, bstar)
            astar = jnp.where(found, aat, astar)
            return (run + tot, bstar, astar)

        _, bstar, astar = lax.fori_loop(
            0, 16, sc, (jnp.int32(0), jnp.int32(0), jnp.int32(0)))
        return bstar, kcur - astar

    zero_hist()

    def p1(ch, _):
        u = data_v[pl.ds(ch * 16, 16)]
        bkey = u ^ ((u >> 31) & _MANT) ^ _SIGN
        data_v[pl.ds(ch * 16, 16)] = bkey
        b = (bkey >> 24) & 255
        plsc.addupdate_scatter(hist_v, [lane * 256 + b], ones)
        return 0

    lax.fori_loop(0, _CHUNKS, p1, 0)
    b1, k1 = merge_and_scan(jnp.int32(k0))
    kpart1 = b1 << 24

    zero_hist()

    def p2(ch, w):
        bkey = data_v[pl.ds(ch * 16, 16)]
        match = ((bkey ^ kpart1) >> 24) == 0
        b = (bkey >> 16) & 255
        plsc.addupdate_scatter(hist_v, [lane * 256 + b], ones, mask=match)
        pos = w + jnp.cumsum(match.astype(jnp.int32)) - 1
        plsc.store_scatter(data_v, [pos], bkey, mask=match)
        return w + jnp.sum(match.astype(jnp.int32))

    cnt2 = lax.fori_loop(0, _CHUNKS, p2, jnp.int32(0))
    b2, k2 = merge_and_scan(k1)
    kpart2 = kpart1 | (b2 << 16)

    zero_hist()

    def p3(ch, w):
        base = ch * 16
        bkey = data_v[pl.ds(base, 16)]
        match = ((base + lane) < cnt2) & (((bkey ^ kpart2) >> 16) == 0)
        b = (bkey >> 8) & 255
        plsc.addupdate_scatter(hist_v, [lane * 256 + b], ones, mask=match)
        pos = w + jnp.cumsum(match.astype(jnp.int32)) - 1
        plsc.store_scatter(data_v, [pos], bkey, mask=match)
        return w + jnp.sum(match.astype(jnp.int32))

    cnt3 = lax.fori_loop(0, (cnt2 + 15) >> 4, p3, jnp.int32(0))
    b3, k3 = merge_and_scan(k2)
    kpart3 = kpart2 | (b3 << 8)

    zero_hist()

    def p4(ch, _):
        base = ch * 16
        bkey = data_v[pl.ds(base, 16)]
        match = ((base + lane) < cnt3) & (((bkey ^ kpart3) >> 8) == 0)
        b = bkey & 255
        plsc.addupdate_scatter(hist_v, [lane * 256 + b], ones, mask=match)
        return 0

    lax.fori_loop(0, (cnt3 + 15) >> 4, p4, 0)
    b4, _ = merge_and_scan(k3)

    bfin = kpart3 | b4
    skey = bfin ^ _SIGN
    ufin = jnp.where(skey >= 0, skey, skey ^ _MANT)
    tvec_v[...] = lax.bitcast_convert_type(
        jnp.broadcast_to(ufin, (16,)), jnp.float32)

    @pl.when((cid == 0) & (sid == 0))
    def _():
        pltpu.sync_copy(tvec_v, t_out)


def _softplus(x):
    return jnp.maximum(x, 0.0) + jnp.log1p(jnp.exp(-jnp.abs(x)))


def _bce_term(x, t):
    return t * jnp.minimum(_softplus(-x), 100.0) + (1.0 - t) * jnp.minimum(
        _softplus(x), 100.0)


def _loss_body(k, n_pos, neg_ref, pos_ref, lab_ref, t_ref, of_ref, oi_ref):
    t_val = t_ref[0]
    neg = lax.bitcast_convert_type(neg_ref[...], jnp.float32)
    sel = neg > t_val
    c = jnp.sum(sel.astype(jnp.int32))
    g = jnp.minimum(_softplus(-neg), 100.0)
    sum_sel = jnp.sum(jnp.where(sel, g, 0.0))
    negneg = jnp.sum(jnp.logical_and(sel, neg < 0.0).astype(jnp.int32))

    g_t = jnp.minimum(_softplus(-t_val), 100.0)
    ties = jnp.int32(k) - c
    neg_bce = (sum_sel + ties.astype(jnp.float32) * g_t) / jnp.float32(k)
    neg_correct = negneg + ties * (t_val < 0.0).astype(jnp.int32)

    x = pos_ref[0:1, :]
    t = lab_ref[0:1, :]
    pos_bce = jnp.sum(_bce_term(x, t)) / jnp.float32(n_pos)
    pos_correct = jnp.sum((x >= 0.0).astype(jnp.int32))

    classify = 0.5 * pos_bce + 0.5 * neg_bce
    loss = classify
    for i in range(1, 5):
        d = pos_ref[i:i + 1, :] - lab_ref[i:i + 1, :]
        ad = jnp.abs(d)
        rl = jnp.sum(jnp.where(ad < 1.0, 0.5 * d * d, ad - 0.5)) / jnp.float32(
            n_pos)
        of_ref[1 + i] = rl
        loss = loss + rl
    of_ref[0] = loss
    of_ref[1] = classify
    oi_ref[0] = pos_correct
    oi_ref[1] = neg_correct


def kernel(pos_output, pos_labels, neg_output, neg_labels):
    del neg_labels
    n_pos = pos_output.shape[0]
    k = min(_NUM_HARD * max(n_pos, 1), neg_output.shape[0])

    n = neg_output.shape[0]
    pad = _N_PAD - n
    negp = lax.bitcast_convert_type(
        jnp.concatenate([neg_output, jnp.full((pad,), -jnp.inf, jnp.float32)]),
        jnp.int32)

    mesh = plsc.VectorSubcoreMesh(core_axis_name="c", subcore_axis_name="s")
    sc_select = functools.partial(
        pl.kernel,
        out_type=jax.ShapeDtypeStruct((16,), jnp.float32),
        mesh=mesh,
        compiler_params=pltpu.CompilerParams(needs_layout_passes=False),
        scratch_types=[
            pltpu.VMEM((_PER_TILE,), jnp.int32),
            pltpu.VMEM((4096,), jnp.int32),
            pltpu.VMEM((256,), jnp.int32),
            pltpu.VMEM_SHARED((4096,), jnp.int32),
            pltpu.VMEM((4096,), jnp.int32),
            pltpu.VMEM((16,), jnp.float32),
        ],
    )(functools.partial(_sc_select_body, k))
    t_arr = sc_select(negp)

    pos_t = pos_output.T
    lab_t = pos_labels.T

    of, oi = pl.pallas_call(
        functools.partial(_loss_body, k, n_pos),
        out_shape=(
            jax.ShapeDtypeStruct((6,), jnp.float32),
            jax.ShapeDtypeStruct((2,), jnp.int32),
        ),
        in_specs=[
            pl.BlockSpec(memory_space=pltpu.VMEM),
            pl.BlockSpec(memory_space=pltpu.VMEM),
            pl.BlockSpec(memory_space=pltpu.VMEM),
            pl.BlockSpec(memory_space=pltpu.SMEM),
        ],
        out_specs=(
            pl.BlockSpec(memory_space=pltpu.SMEM),
            pl.BlockSpec(memory_space=pltpu.SMEM),
        ),
    )(negp.reshape(1954, 512), pos_t, lab_t, t_arr)

    return (
        of[0], of[1], of[2], of[3], of[4], of[5],
        oi[0],
        jnp.asarray(n_pos, dtype=jnp.int32),
        oi[1],
        jnp.asarray(k, dtype=jnp.int32),
    )


# speedup vs baseline: 1.5930x; 1.5930x over previous
import functools

import jax
import jax.numpy as jnp
from jax import lax
from jax.experimental import pallas as pl
from jax.experimental.pallas import tpu as pltpu
from jax.experimental.pallas import tpu_sc as plsc

_NUM_HARD = 2
_SIGN = -2147483648
_MANT = 2147483647

_N_TILES = 16
_PER_TILE = 62528
_N_PAD = _N_TILES * _PER_TILE
_CHUNKS = _PER_TILE // 16


def _sc_select_body(k0, neg_hbm, t_out, data_v, hist_v, loc_v, shared_v,
                    merge_v, tvec_v):
    cid = lax.axis_index("c")
    sid = lax.axis_index("s")
    pltpu.sync_copy(neg_hbm.at[pl.ds(sid * _PER_TILE, _PER_TILE)], data_v)
    lane = lax.iota(jnp.int32, 16)
    ones = jnp.ones((16,), jnp.int32)
    lane256 = lane * 256

    def zero_hist():
        @plsc.parallel_loop(0, 1024, unroll=8)
        def _(i):
            hist_v[pl.ds(i * 16, 16)] = jnp.zeros((16,), jnp.int32)

    def reduce_sub(src_v, nsub):
        def lr(cb, _):
            def rr(r, acc):
                return acc + src_v[pl.ds(r * 256 + cb * 16, 16)]
            acc = lax.fori_loop(0, nsub, rr, jnp.zeros((16,), jnp.int32))
            loc_v[pl.ds(cb * 16, 16)] = acc
            return 0
        lax.fori_loop(0, 16, lr, 0)

    def merge_and_scan(kcur):
        reduce_sub(hist_v, 64)
        pltpu.sync_copy(loc_v, shared_v.at[pl.ds(sid * 256, 256)])
        plsc.subcore_barrier()
        pltpu.sync_copy(shared_v, merge_v)
        plsc.subcore_barrier()
        reduce_sub(merge_v, 16)

        def sc(i, carry):
            run, bstar, astar = carry
            cc = 15 - i
            h = loc_v[pl.ds(cc * 16, 16)]
            tot = jnp.sum(h)
            above = (run + tot) - jnp.cumsum(h)
            maska = above < kcur
            cntm = jnp.sum(maska.astype(jnp.int32))
            found = cntm > 0
            j0 = 16 - cntm
            aat = jnp.sum(jnp.where(lane == j0, above, 0))
            bstar = jnp.where(found, cc * 16 + j0, bstar)
            astar = jnp.where(found, aat, astar)
            return (run + tot, bstar, astar)

        _, bstar, astar = lax.fori_loop(
            0, 16, sc, (jnp.int32(0), jnp.int32(0), jnp.int32(0)))
        return bstar, kcur - astar

    zero_hist()

    @plsc.parallel_loop(0, _CHUNKS, unroll=4)
    def _(ch):
        u = data_v[pl.ds(ch * 16, 16)]
        bkey = u ^ ((u >> 31) & _MANT) ^ _SIGN
        data_v[pl.ds(ch * 16, 16)] = bkey
        b = (bkey >> 24) & 255
        plsc.addupdate_scatter(
            hist_v, [(ch & 3) * 4096 + lane256 + b], ones)

    b1, k1 = merge_and_scan(jnp.int32(k0))
    kpart1 = b1 << 24

    zero_hist()

    @plsc.parallel_loop(0, _CHUNKS, unroll=4)
    def _(ch):
        bkey = data_v[pl.ds(ch * 16, 16)]
        match = ((bkey ^ kpart1) >> 24) == 0
        b = (bkey >> 16) & 255
        plsc.addupdate_scatter(
            hist_v, [(ch & 3) * 4096 + lane256 + b], ones, mask=match)

    b2, k2 = merge_and_scan(k1)
    kpart2 = kpart1 | (b2 << 16)

    zero_hist()

    @plsc.parallel_loop(0, _CHUNKS, unroll=4)
    def _(ch):
        bkey = data_v[pl.ds(ch * 16, 16)]
        match = ((bkey ^ kpart2) >> 16) == 0
        b = (bkey >> 8) & 255
        plsc.addupdate_scatter(
            hist_v, [(ch & 3) * 4096 + lane256 + b], ones, mask=match)

    b3, k3 = merge_and_scan(k2)
    kpart3 = kpart2 | (b3 << 8)

    zero_hist()

    @plsc.parallel_loop(0, _CHUNKS, unroll=4)
    def _(ch):
        bkey = data_v[pl.ds(ch * 16, 16)]
        match = ((bkey ^ kpart3) >> 8) == 0
        b = bkey & 255
        plsc.addupdate_scatter(
            hist_v, [(ch & 3) * 4096 + lane256 + b], ones, mask=match)

    b4, _ = merge_and_scan(k3)

    bfin = kpart3 | b4
    skey = bfin ^ _SIGN
    ufin = jnp.where(skey >= 0, skey, skey ^ _MANT)
    tvec_v[...] = lax.bitcast_convert_type(
        jnp.broadcast_to(ufin, (16,)), jnp.float32)

    @pl.when((cid == 0) & (sid == 0))
    def _():
        pltpu.sync_copy(tvec_v, t_out)


def _softplus(x):
    return jnp.maximum(x, 0.0) + jnp.log1p(jnp.exp(-jnp.abs(x)))


def _bce_term(x, t):
    return t * jnp.minimum(_softplus(-x), 100.0) + (1.0 - t) * jnp.minimum(
        _softplus(x), 100.0)


def _loss_body(k, n_pos, neg_ref, pos_ref, lab_ref, t_ref, of_ref, oi_ref):
    t_val = t_ref[0]
    neg = lax.bitcast_convert_type(neg_ref[...], jnp.float32)
    sel = neg > t_val
    c = jnp.sum(sel.astype(jnp.int32))
    g = jnp.minimum(_softplus(-neg), 100.0)
    sum_sel = jnp.sum(jnp.where(sel, g, 0.0))
    negneg = jnp.sum(jnp.logical_and(sel, neg < 0.0).astype(jnp.int32))

    g_t = jnp.minimum(_softplus(-t_val), 100.0)
    ties = jnp.int32(k) - c
    neg_bce = (sum_sel + ties.astype(jnp.float32) * g_t) / jnp.float32(k)
    neg_correct = negneg + ties * (t_val < 0.0).astype(jnp.int32)

    x = pos_ref[0:1, :]
    t = lab_ref[0:1, :]
    pos_bce = jnp.sum(_bce_term(x, t)) / jnp.float32(n_pos)
    pos_correct = jnp.sum((x >= 0.0).astype(jnp.int32))

    classify = 0.5 * pos_bce + 0.5 * neg_bce
    loss = classify
    for i in range(1, 5):
        d = pos_ref[i:i + 1, :] - lab_ref[i:i + 1, :]
        ad = jnp.abs(d)
        rl = jnp.sum(jnp.where(ad < 1.0, 0.5 * d * d, ad - 0.5)) / jnp.float32(
            n_pos)
        of_ref[1 + i] = rl
        loss = loss + rl
    of_ref[0] = loss
    of_ref[1] = classify
    oi_ref[0] = pos_correct
    oi_ref[1] = neg_correct


def kernel(pos_output, pos_labels, neg_output, neg_labels):
    del neg_labels
    n_pos = pos_output.shape[0]
    k = min(_NUM_HARD * max(n_pos, 1), neg_output.shape[0])

    n = neg_output.shape[0]
    pad = _N_PAD - n
    negp = lax.bitcast_convert_type(
        jnp.concatenate([neg_output, jnp.full((pad,), -jnp.inf, jnp.float32)]),
        jnp.int32)

    mesh = plsc.VectorSubcoreMesh(core_axis_name="c", subcore_axis_name="s")
    sc_select = functools.partial(
        pl.kernel,
        out_type=jax.ShapeDtypeStruct((16,), jnp.float32),
        mesh=mesh,
        compiler_params=pltpu.CompilerParams(needs_layout_passes=False),
        scratch_types=[
            pltpu.VMEM((_PER_TILE,), jnp.int32),
            pltpu.VMEM((16384,), jnp.int32),
            pltpu.VMEM((256,), jnp.int32),
            pltpu.VMEM_SHARED((4096,), jnp.int32),
            pltpu.VMEM((4096,), jnp.int32),
            pltpu.VMEM((16,), jnp.float32),
        ],
    )(functools.partial(_sc_select_body, k))
    t_arr = sc_select(negp)

    pos_t = pos_output.T
    lab_t = pos_labels.T

    of, oi = pl.pallas_call(
        functools.partial(_loss_body, k, n_pos),
        out_shape=(
            jax.ShapeDtypeStruct((6,), jnp.float32),
            jax.ShapeDtypeStruct((2,), jnp.int32),
        ),
        in_specs=[
            pl.BlockSpec(memory_space=pltpu.VMEM),
            pl.BlockSpec(memory_space=pltpu.VMEM),
            pl.BlockSpec(memory_space=pltpu.VMEM),
            pl.BlockSpec(memory_space=pltpu.SMEM),
        ],
        out_specs=(
            pl.BlockSpec(memory_space=pltpu.SMEM),
            pl.BlockSpec(memory_space=pltpu.SMEM),
        ),
    )(negp.reshape(1954, 512), pos_t, lab_t, t_arr)

    return (
        of[0], of[1], of[2], of[3], of[4], of[5],
        oi[0],
        jnp.asarray(n_pos, dtype=jnp.int32),
        oi[1],
        jnp.asarray(k, dtype=jnp.int32),
    )
